# baseline (speedup 1.0000x reference)
import functools

import jax
import jax.numpy as jnp
from jax import lax
from jax.experimental import pallas as pl
from jax.experimental.pallas import tpu as pltpu
from jax.experimental.pallas import tpu_sc as plsc

BATCH = 16384
EMBED_DIM = 64
NUM_ROWS = 1000000

_NC = 2
_NS = 16
_NW = _NC * _NS
_W = BATCH // _NW
_CH = 128
_NCH = _W // _CH

_mesh = plsc.VectorSubcoreMesh(core_axis_name="c", subcore_axis_name="s")


@functools.partial(
    pl.kernel,
    mesh=_mesh,
    out_type=(
        jax.ShapeDtypeStruct((BATCH, EMBED_DIM), jnp.float32),
        jax.ShapeDtypeStruct((BATCH, EMBED_DIM), jnp.float32),
    ),
    scratch_types=[
        pltpu.VMEM((_W,), jnp.int32),
        pltpu.VMEM((_NCH, _CH), jnp.int32),
        pltpu.VMEM((_CH, 2 * EMBED_DIM), jnp.float32),
        pltpu.VMEM((_CH, 2 * EMBED_DIM), jnp.float32),
        pltpu.VMEM((_W, EMBED_DIM), jnp.float32),
        pltpu.SemaphoreType.DMA,
        pltpu.SemaphoreType.DMA,
    ],
    compiler_params=pltpu.CompilerParams(needs_layout_passes=False),
)
def _two_tower_gather(uid_hbm, iid_hbm, utab_hbm, itab_hbm,
                      uout_hbm, iout_hbm,
                      ids_v, pidx_v, pairs0_v, pairs1_v, rows_v, sem0, sem1):
    wid = lax.axis_index("s") * _NC + lax.axis_index("c")
    base = wid * _W
    lane = lax.iota(jnp.int32, 16)
    one16 = jnp.full((16,), 1, jnp.int32)
    six16 = jnp.full((16,), 6, jnp.int32)

    for id_hbm, tab, out_hbm in (
        (uid_hbm, utab_hbm, uout_hbm),
        (iid_hbm, itab_hbm, iout_hbm),
    ):
        pltpu.sync_copy(id_hbm.at[wid], ids_v)

        for j in range(_NCH):
            for k in range(_CH // 16):
                v16 = ids_v[pl.ds(j * _CH + k * 16, 16)]
                pidx_v[j, pl.ds(k * 16, 16)] = lax.shift_right_logical(v16, 1)

        def fire(j, buf, sem):
            pltpu.async_copy(tab.at[pidx_v.at[j]], buf, sem)

        def drain(j, buf, sem):
            pltpu.make_async_copy(tab.at[pidx_v.at[j]], buf, sem).wait()

        def extract(j, buf):
            for k in range(_CH // 16):
                i16 = lane + j * _CH + k * 16
                ids16 = ids_v[pl.ds(j * _CH + k * 16, 16)]
                off16 = lax.shift_left(
                    lax.bitwise_and(ids16, one16), six16)
                iloc16 = lane + k * 16
                for c in range(EMBED_DIM):
                    c16 = jnp.full((16,), c, jnp.int32)
                    vals = plsc.load_gather(buf, [iloc16, off16 + c16])
                    plsc.store_scatter(rows_v, [i16, c16], vals)

        fire(0, pairs0_v, sem0)

        @pl.loop(0, _NCH, step=2)
        def _chunk(g):
            @pl.when(g + 1 < _NCH)
            def _():
                fire(g + 1, pairs1_v, sem1)

            drain(g, pairs0_v, sem0)
            extract(g, pairs0_v)

            @pl.when(g + 2 < _NCH)
            def _():
                fire(g + 2, pairs0_v, sem0)

            @pl.when(g + 1 < _NCH)
            def _():
                drain(g + 1, pairs1_v, sem1)
                extract(g + 1, pairs1_v)

        pltpu.sync_copy(rows_v, out_hbm.at[pl.ds(base, _W)])


def kernel(user_ids, pos_item_ids, user_table, item_table):
    u2 = user_ids.astype(jnp.int32).reshape(_NW, _W)
    i2 = pos_item_ids.astype(jnp.int32).reshape(_NW, _W)
    ut2 = user_table.reshape(NUM_ROWS // 2, 2 * EMBED_DIM)
    it2 = item_table.reshape(NUM_ROWS // 2, 2 * EMBED_DIM)
    return _two_tower_gather(u2, i2, ut2, it2)


# speedup vs baseline: 1.2517x; 1.2517x over previous
import functools

import jax
import jax.numpy as jnp
from jax import lax
from jax.experimental import pallas as pl
from jax.experimental.pallas import tpu as pltpu
from jax.experimental.pallas import tpu_sc as plsc

BATCH = 16384
EMBED_DIM = 64
NUM_ROWS = 1000000

_NC = 2
_NS = 16
_NW = _NC * _NS
_W = BATCH // _NW
_CH = 128
_NCH = _W // _CH

_TCOLS = 2048
_TGRID = (NUM_ROWS + _TCOLS - 1) // _TCOLS

_mesh = plsc.VectorSubcoreMesh(core_axis_name="c", subcore_axis_name="s")


_PROWS = _TGRID * (_TCOLS // 2)


def _relayout_block(in_ref, o_ref):
    x = in_ref[...]
    xt = jnp.swapaxes(x, 0, 1)
    o_ref[:, 0:EMBED_DIM] = xt[0:_TCOLS // 2]
    o_ref[:, EMBED_DIM:2 * EMBED_DIM] = xt[_TCOLS // 2:_TCOLS]


_relayout = pl.pallas_call(
    _relayout_block,
    grid=(_TGRID,),
    in_specs=[pl.BlockSpec((EMBED_DIM, _TCOLS), lambda i: (0, i))],
    out_specs=pl.BlockSpec((_TCOLS // 2, 2 * EMBED_DIM), lambda i: (i, 0)),
    out_shape=jax.ShapeDtypeStruct((_PROWS, 2 * EMBED_DIM), jnp.float32),
)


@functools.partial(
    pl.kernel,
    mesh=_mesh,
    out_type=jax.ShapeDtypeStruct((BATCH, EMBED_DIM), jnp.float32),
    scratch_types=[
        pltpu.VMEM((_W,), jnp.int32),
        pltpu.VMEM((_NCH, _CH), jnp.int32),
        pltpu.VMEM((_CH, 2 * EMBED_DIM), jnp.float32),
        pltpu.VMEM((_CH, 2 * EMBED_DIM), jnp.float32),
        pltpu.VMEM((_W, EMBED_DIM), jnp.float32),
        pltpu.SemaphoreType.DMA,
        pltpu.SemaphoreType.DMA,
    ],
    compiler_params=pltpu.CompilerParams(needs_layout_passes=False),
)
def _tower_gather(id_hbm, tab, out_hbm,
                  ids_v, pidx_v, pairs0_v, pairs1_v, rows_v, sem0, sem1):
    wid = lax.axis_index("s") * _NC + lax.axis_index("c")
    base = wid * _W
    lane = lax.iota(jnp.int32, 16)
    one16 = jnp.full((16,), 1, jnp.int32)
    six16 = jnp.full((16,), 6, jnp.int32)

    pltpu.sync_copy(id_hbm.at[wid], ids_v)

    k10 = jnp.full((16,), 10, jnp.int32)
    m1023 = jnp.full((16,), 1023, jnp.int32)
    for j in range(_NCH):
        for k in range(_CH // 16):
            v16 = ids_v[pl.ds(j * _CH + k * 16, 16)]
            pidx_v[j, pl.ds(k * 16, 16)] = lax.shift_left(
                lax.shift_right_logical(v16, 11), k10
            ) + lax.bitwise_and(v16, m1023)

    def fire(j, buf, sem):
        pltpu.async_copy(tab.at[pidx_v.at[j]], buf, sem)

    def drain(j, buf, sem):
        pltpu.make_async_copy(tab.at[pidx_v.at[j]], buf, sem).wait()

    def extract(j, buf):
        for k in range(_CH // 16):
            i16 = lane + j * _CH + k * 16
            ids16 = ids_v[pl.ds(j * _CH + k * 16, 16)]
            off16 = lax.shift_left(
                lax.bitwise_and(lax.shift_right_logical(ids16, 10), one16),
                six16)
            iloc16 = lane + k * 16
            for c in range(EMBED_DIM):
                c16 = jnp.full((16,), c, jnp.int32)
                vals = plsc.load_gather(buf, [iloc16, off16 + c16])
                plsc.store_scatter(rows_v, [i16, c16], vals)

    fire(0, pairs0_v, sem0)

    @pl.loop(0, _NCH, step=2)
    def _chunk(g):
        @pl.when(g + 1 < _NCH)
        def _():
            fire(g + 1, pairs1_v, sem1)

        drain(g, pairs0_v, sem0)
        extract(g, pairs0_v)

        @pl.when(g + 2 < _NCH)
        def _():
            fire(g + 2, pairs0_v, sem0)

        @pl.when(g + 1 < _NCH)
        def _():
            drain(g + 1, pairs1_v, sem1)
            extract(g + 1, pairs1_v)

    pltpu.sync_copy(rows_v, out_hbm.at[pl.ds(base, _W)])


def kernel(user_ids, pos_item_ids, user_table, item_table):
    u2 = user_ids.astype(jnp.int32).reshape(_NW, _W)
    i2 = pos_item_ids.astype(jnp.int32).reshape(_NW, _W)
    ut2 = _relayout(user_table.T)
    user_emb = _tower_gather(u2, ut2)
    it2 = _relayout(item_table.T)
    item_emb = _tower_gather(i2, it2)
    return (user_emb, item_emb)


# speedup vs baseline: 2.0600x; 1.6457x over previous
import functools

import jax
import jax.numpy as jnp
from jax import lax
from jax.experimental import pallas as pl
from jax.experimental.pallas import tpu as pltpu
from jax.experimental.pallas import tpu_sc as plsc

BATCH = 16384
EMBED_DIM = 64
NUM_ROWS = 1000000

_NC = 2
_NS = 16
_NW = _NC * _NS
_W = BATCH // _NW
_CH = 128
_NCH = _W // _CH

_TCOLS = 8192
_HALF = _TCOLS // 2
_TGRID = (NUM_ROWS + _TCOLS - 1) // _TCOLS

_mesh = plsc.VectorSubcoreMesh(core_axis_name="c", subcore_axis_name="s")


_PROWS = _TGRID * (_TCOLS // 2)


def _relayout_block(in_ref, o_ref):
    x = in_ref[...]
    xt = jnp.swapaxes(x, 0, 1)
    o_ref[:, 0:EMBED_DIM] = xt[0:_HALF]
    o_ref[:, EMBED_DIM:2 * EMBED_DIM] = xt[_HALF:_TCOLS]


_relayout = pl.pallas_call(
    _relayout_block,
    grid=(_TGRID,),
    in_specs=[pl.BlockSpec((EMBED_DIM, _TCOLS), lambda i: (0, i))],
    out_specs=pl.BlockSpec((_TCOLS // 2, 2 * EMBED_DIM), lambda i: (i, 0)),
    out_shape=jax.ShapeDtypeStruct((_PROWS, 2 * EMBED_DIM), jnp.float32),
)


@functools.partial(
    pl.kernel,
    mesh=_mesh,
    out_type=jax.ShapeDtypeStruct((BATCH, EMBED_DIM), jnp.float32),
    scratch_types=[
        pltpu.VMEM((_W,), jnp.int32),
        pltpu.VMEM((_NCH, _CH), jnp.int32),
        pltpu.VMEM((_CH, 2 * EMBED_DIM), jnp.float32),
        pltpu.VMEM((_CH, 2 * EMBED_DIM), jnp.float32),
        pltpu.VMEM((_W, EMBED_DIM), jnp.float32),
        pltpu.SemaphoreType.DMA,
        pltpu.SemaphoreType.DMA,
    ],
    compiler_params=pltpu.CompilerParams(needs_layout_passes=False),
)
def _tower_gather(id_hbm, tab, out_hbm,
                  ids_v, pidx_v, pairs0_v, pairs1_v, rows_v, sem0, sem1):
    wid = lax.axis_index("s") * _NC + lax.axis_index("c")
    base = wid * _W
    lane = lax.iota(jnp.int32, 16)
    one16 = jnp.full((16,), 1, jnp.int32)
    six16 = jnp.full((16,), 6, jnp.int32)

    pltpu.sync_copy(id_hbm.at[wid], ids_v)

    khalf = jnp.full((16,), _HALF.bit_length() - 1, jnp.int32)
    mhalf = jnp.full((16,), _HALF - 1, jnp.int32)
    for j in range(_NCH):
        for k in range(_CH // 16):
            v16 = ids_v[pl.ds(j * _CH + k * 16, 16)]
            pidx_v[j, pl.ds(k * 16, 16)] = lax.shift_left(
                lax.shift_right_logical(v16, _TCOLS.bit_length() - 1), khalf
            ) + lax.bitwise_and(v16, mhalf)

    def fire(j, buf, sem):
        pltpu.async_copy(tab.at[pidx_v.at[j]], buf, sem)

    def drain(j, buf, sem):
        pltpu.make_async_copy(tab.at[pidx_v.at[j]], buf, sem).wait()

    def extract(j, buf):
        for k in range(_CH // 16):
            i16 = lane + j * _CH + k * 16
            ids16 = ids_v[pl.ds(j * _CH + k * 16, 16)]
            off16 = lax.shift_left(
                lax.bitwise_and(
                    lax.shift_right_logical(
                        ids16, _HALF.bit_length() - 1), one16),
                six16)
            iloc16 = lane + k * 16
            for c in range(EMBED_DIM):
                c16 = jnp.full((16,), c, jnp.int32)
                vals = plsc.load_gather(buf, [iloc16, off16 + c16])
                plsc.store_scatter(rows_v, [i16, c16], vals)

    fire(0, pairs0_v, sem0)

    @pl.loop(0, _NCH, step=2)
    def _chunk(g):
        @pl.when(g + 1 < _NCH)
        def _():
            fire(g + 1, pairs1_v, sem1)

        drain(g, pairs0_v, sem0)
        extract(g, pairs0_v)

        @pl.when(g + 2 < _NCH)
        def _():
            fire(g + 2, pairs0_v, sem0)

        @pl.when(g + 1 < _NCH)
        def _():
            drain(g + 1, pairs1_v, sem1)
            extract(g + 1, pairs1_v)

    pltpu.sync_copy(rows_v, out_hbm.at[pl.ds(base, _W)])


def kernel(user_ids, pos_item_ids, user_table, item_table):
    u2 = user_ids.astype(jnp.int32).reshape(_NW, _W)
    i2 = pos_item_ids.astype(jnp.int32).reshape(_NW, _W)
    ut2 = _relayout(user_table.T)
    user_emb = _tower_gather(u2, ut2)
    it2 = _relayout(item_table.T)
    item_emb = _tower_gather(i2, it2)
    return (user_emb, item_emb)


# speedup vs baseline: 2.3120x; 1.1223x over previous
import functools

import jax
import jax.numpy as jnp
from jax import lax
from jax.experimental import pallas as pl
from jax.experimental.pallas import tpu as pltpu
from jax.experimental.pallas import tpu_sc as plsc

BATCH = 16384
EMBED_DIM = 64
NUM_ROWS = 1000000

_NC = 2
_NS = 16
_NW = _NC * _NS
_W = BATCH // _NW
_CH = 128
_NCH = _W // _CH

_TCOLS = 16384
_HALF = _TCOLS // 2
_TGRID = (NUM_ROWS + _TCOLS - 1) // _TCOLS

_mesh = plsc.VectorSubcoreMesh(core_axis_name="c", subcore_axis_name="s")


_PROWS = _TGRID * (_TCOLS // 2)


def _relayout_block(in_ref, o_ref):
    x = in_ref[...]
    xt = jnp.swapaxes(x, 0, 1)
    o_ref[:, 0:EMBED_DIM] = xt[0:_HALF]
    o_ref[:, EMBED_DIM:2 * EMBED_DIM] = xt[_HALF:_TCOLS]


_relayout = pl.pallas_call(
    _relayout_block,
    grid=(_TGRID,),
    in_specs=[pl.BlockSpec((EMBED_DIM, _TCOLS), lambda i: (0, i))],
    out_specs=pl.BlockSpec((_TCOLS // 2, 2 * EMBED_DIM), lambda i: (i, 0)),
    out_shape=jax.ShapeDtypeStruct((_PROWS, 2 * EMBED_DIM), jnp.float32),
)


@functools.partial(
    pl.kernel,
    mesh=_mesh,
    out_type=jax.ShapeDtypeStruct((BATCH, EMBED_DIM), jnp.float32),
    scratch_types=[
        pltpu.VMEM((_W,), jnp.int32),
        pltpu.VMEM((_NCH, _CH), jnp.int32),
        pltpu.VMEM((_CH, 2 * EMBED_DIM), jnp.float32),
        pltpu.VMEM((_CH, 2 * EMBED_DIM), jnp.float32),
        pltpu.VMEM((_W, EMBED_DIM), jnp.float32),
        pltpu.SemaphoreType.DMA,
        pltpu.SemaphoreType.DMA,
    ],
    compiler_params=pltpu.CompilerParams(needs_layout_passes=False),
)
def _tower_gather(id_hbm, tab, out_hbm,
                  ids_v, pidx_v, pairs0_v, pairs1_v, rows_v, sem0, sem1):
    wid = lax.axis_index("s") * _NC + lax.axis_index("c")
    base = wid * _W
    lane = lax.iota(jnp.int32, 16)
    one16 = jnp.full((16,), 1, jnp.int32)
    six16 = jnp.full((16,), 6, jnp.int32)

    pltpu.sync_copy(id_hbm.at[wid], ids_v)

    khalf = jnp.full((16,), _HALF.bit_length() - 1, jnp.int32)
    mhalf = jnp.full((16,), _HALF - 1, jnp.int32)
    for j in range(_NCH):
        for k in range(_CH // 16):
            v16 = ids_v[pl.ds(j * _CH + k * 16, 16)]
            pidx_v[j, pl.ds(k * 16, 16)] = lax.shift_left(
                lax.shift_right_logical(v16, _TCOLS.bit_length() - 1), khalf
            ) + lax.bitwise_and(v16, mhalf)

    def fire(j, buf, sem):
        pltpu.async_copy(tab.at[pidx_v.at[j]], buf, sem)

    def drain(j, buf, sem):
        pltpu.make_async_copy(tab.at[pidx_v.at[j]], buf, sem).wait()

    def extract(j, buf):
        for k in range(_CH // 16):
            i16 = lane + j * _CH + k * 16
            ids16 = ids_v[pl.ds(j * _CH + k * 16, 16)]
            off16 = lax.shift_left(
                lax.bitwise_and(
                    lax.shift_right_logical(
                        ids16, _HALF.bit_length() - 1), one16),
                six16)
            iloc16 = lane + k * 16
            for c in range(EMBED_DIM):
                c16 = jnp.full((16,), c, jnp.int32)
                vals = plsc.load_gather(buf, [iloc16, off16 + c16])
                plsc.store_scatter(rows_v, [i16, c16], vals)

    fire(0, pairs0_v, sem0)

    @pl.loop(0, _NCH, step=2)
    def _chunk(g):
        @pl.when(g + 1 < _NCH)
        def _():
            fire(g + 1, pairs1_v, sem1)

        drain(g, pairs0_v, sem0)
        extract(g, pairs0_v)

        @pl.when(g + 2 < _NCH)
        def _():
            fire(g + 2, pairs0_v, sem0)

        @pl.when(g + 1 < _NCH)
        def _():
            drain(g + 1, pairs1_v, sem1)
            extract(g + 1, pairs1_v)

    pltpu.sync_copy(rows_v, out_hbm.at[pl.ds(base, _W)])


def kernel(user_ids, pos_item_ids, user_table, item_table):
    u2 = user_ids.astype(jnp.int32).reshape(_NW, _W)
    i2 = pos_item_ids.astype(jnp.int32).reshape(_NW, _W)
    ut2 = _relayout(user_table.T)
    user_emb = _tower_gather(u2, ut2)
    it2 = _relayout(item_table.T)
    item_emb = _tower_gather(i2, it2)
    return (user_emb, item_emb)


# speedup vs baseline: 2.4430x; 1.0567x over previous
import functools

import jax
import jax.numpy as jnp
from jax import lax
from jax.experimental import pallas as pl
from jax.experimental.pallas import tpu as pltpu
from jax.experimental.pallas import tpu_sc as plsc

BATCH = 16384
EMBED_DIM = 64
NUM_ROWS = 1000000

_NC = 2
_NS = 16
_NW = _NC * _NS
_W = BATCH // _NW
_CH = 128
_NCH = _W // _CH

_TCOLS = 32768
_HALF = _TCOLS // 2
_TGRID = (NUM_ROWS + _TCOLS - 1) // _TCOLS

_mesh = plsc.VectorSubcoreMesh(core_axis_name="c", subcore_axis_name="s")


_PROWS = _TGRID * (_TCOLS // 2)


def _relayout_block(in_ref, o_ref):
    x = in_ref[...]
    xt = jnp.swapaxes(x, 0, 1)
    o_ref[:, 0:EMBED_DIM] = xt[0:_HALF]
    o_ref[:, EMBED_DIM:2 * EMBED_DIM] = xt[_HALF:_TCOLS]


_relayout = pl.pallas_call(
    _relayout_block,
    grid=(_TGRID,),
    in_specs=[pl.BlockSpec((EMBED_DIM, _TCOLS), lambda i: (0, i))],
    out_specs=pl.BlockSpec((_TCOLS // 2, 2 * EMBED_DIM), lambda i: (i, 0)),
    out_shape=jax.ShapeDtypeStruct((_PROWS, 2 * EMBED_DIM), jnp.float32),
)


@functools.partial(
    pl.kernel,
    mesh=_mesh,
    out_type=jax.ShapeDtypeStruct((BATCH, EMBED_DIM), jnp.float32),
    scratch_types=[
        pltpu.VMEM((_W,), jnp.int32),
        pltpu.VMEM((_NCH, _CH), jnp.int32),
        pltpu.VMEM((_CH, 2 * EMBED_DIM), jnp.float32),
        pltpu.VMEM((_CH, 2 * EMBED_DIM), jnp.float32),
        pltpu.VMEM((_W, EMBED_DIM), jnp.float32),
        pltpu.SemaphoreType.DMA,
        pltpu.SemaphoreType.DMA,
    ],
    compiler_params=pltpu.CompilerParams(needs_layout_passes=False),
)
def _tower_gather(id_hbm, tab, out_hbm,
                  ids_v, pidx_v, pairs0_v, pairs1_v, rows_v, sem0, sem1):
    wid = lax.axis_index("s") * _NC + lax.axis_index("c")
    base = wid * _W
    lane = lax.iota(jnp.int32, 16)
    one16 = jnp.full((16,), 1, jnp.int32)
    six16 = jnp.full((16,), 6, jnp.int32)

    pltpu.sync_copy(id_hbm.at[wid], ids_v)

    khalf = jnp.full((16,), _HALF.bit_length() - 1, jnp.int32)
    mhalf = jnp.full((16,), _HALF - 1, jnp.int32)
    for j in range(_NCH):
        for k in range(_CH // 16):
            v16 = ids_v[pl.ds(j * _CH + k * 16, 16)]
            pidx_v[j, pl.ds(k * 16, 16)] = lax.shift_left(
                lax.shift_right_logical(v16, _TCOLS.bit_length() - 1), khalf
            ) + lax.bitwise_and(v16, mhalf)

    def fire(j, buf, sem):
        pltpu.async_copy(tab.at[pidx_v.at[j]], buf, sem)

    def drain(j, buf, sem):
        pltpu.make_async_copy(tab.at[pidx_v.at[j]], buf, sem).wait()

    def extract(j, buf):
        for k in range(_CH // 16):
            i16 = lane + j * _CH + k * 16
            ids16 = ids_v[pl.ds(j * _CH + k * 16, 16)]
            off16 = lax.shift_left(
                lax.bitwise_and(
                    lax.shift_right_logical(
                        ids16, _HALF.bit_length() - 1), one16),
                six16)
            iloc16 = lane + k * 16
            for c in range(EMBED_DIM):
                c16 = jnp.full((16,), c, jnp.int32)
                vals = plsc.load_gather(buf, [iloc16, off16 + c16])
                plsc.store_scatter(rows_v, [i16, c16], vals)

    fire(0, pairs0_v, sem0)

    @pl.loop(0, _NCH, step=2)
    def _chunk(g):
        @pl.when(g + 1 < _NCH)
        def _():
            fire(g + 1, pairs1_v, sem1)

        drain(g, pairs0_v, sem0)
        extract(g, pairs0_v)

        @pl.when(g + 2 < _NCH)
        def _():
            fire(g + 2, pairs0_v, sem0)

        @pl.when(g + 1 < _NCH)
        def _():
            drain(g + 1, pairs1_v, sem1)
            extract(g + 1, pairs1_v)

    pltpu.sync_copy(rows_v, out_hbm.at[pl.ds(base, _W)])


def kernel(user_ids, pos_item_ids, user_table, item_table):
    u2 = user_ids.astype(jnp.int32).reshape(_NW, _W)
    i2 = pos_item_ids.astype(jnp.int32).reshape(_NW, _W)
    ut2 = _relayout(user_table.T)
    user_emb = _tower_gather(u2, ut2)
    it2 = _relayout(item_table.T)
    item_emb = _tower_gather(i2, it2)
    return (user_emb, item_emb)


# speedup vs baseline: 2.5464x; 1.0423x over previous
import functools

import jax
import jax.numpy as jnp
from jax import lax
from jax.experimental import pallas as pl
from jax.experimental.pallas import tpu as pltpu
from jax.experimental.pallas import tpu_sc as plsc

BATCH = 16384
EMBED_DIM = 64
NUM_ROWS = 1000000

_NC = 2
_NS = 16
_NW = _NC * _NS
_W = BATCH // _NW
_CH = 128
_NCH = _W // _CH

_TCOLS = 32768
_HALF = _TCOLS // 2
_TGRID = (NUM_ROWS + _TCOLS - 1) // _TCOLS

_mesh = plsc.VectorSubcoreMesh(core_axis_name="c", subcore_axis_name="s")


_PROWS = _TGRID * (_TCOLS // 2)


def _relayout_block(in_ref, o_ref):
    x = in_ref[...]
    xt = jnp.swapaxes(x, 0, 1)
    o_ref[:, 0:EMBED_DIM] = xt[0:_HALF]
    o_ref[:, EMBED_DIM:2 * EMBED_DIM] = xt[_HALF:_TCOLS]


_relayout = pl.pallas_call(
    _relayout_block,
    grid=(_TGRID,),
    in_specs=[pl.BlockSpec((EMBED_DIM, _TCOLS), lambda i: (0, i))],
    out_specs=pl.BlockSpec((_TCOLS // 2, 2 * EMBED_DIM), lambda i: (i, 0)),
    out_shape=jax.ShapeDtypeStruct((_PROWS, 2 * EMBED_DIM), jnp.float32),
)


@functools.partial(
    pl.kernel,
    mesh=_mesh,
    out_type=jax.ShapeDtypeStruct((EMBED_DIM, BATCH), jnp.float32),
    scratch_types=[
        pltpu.VMEM((_W,), jnp.int32),
        pltpu.VMEM((_NCH, _CH), jnp.int32),
        pltpu.VMEM((_CH, 2 * EMBED_DIM), jnp.float32),
        pltpu.VMEM((_CH, 2 * EMBED_DIM), jnp.float32),
        pltpu.VMEM((EMBED_DIM, _W), jnp.float32),
        pltpu.SemaphoreType.DMA,
        pltpu.SemaphoreType.DMA,
    ],
    compiler_params=pltpu.CompilerParams(needs_layout_passes=False),
)
def _tower_gather(id_hbm, tab, out_hbm,
                  ids_v, pidx_v, pairs0_v, pairs1_v, rows_v, sem0, sem1):
    wid = lax.axis_index("s") * _NC + lax.axis_index("c")
    base = wid * _W
    lane = lax.iota(jnp.int32, 16)
    one16 = jnp.full((16,), 1, jnp.int32)
    six16 = jnp.full((16,), 6, jnp.int32)

    pltpu.sync_copy(id_hbm.at[wid], ids_v)

    khalf = jnp.full((16,), _HALF.bit_length() - 1, jnp.int32)
    mhalf = jnp.full((16,), _HALF - 1, jnp.int32)
    for j in range(_NCH):
        for k in range(_CH // 16):
            v16 = ids_v[pl.ds(j * _CH + k * 16, 16)]
            pidx_v[j, pl.ds(k * 16, 16)] = lax.shift_left(
                lax.shift_right_logical(v16, _TCOLS.bit_length() - 1), khalf
            ) + lax.bitwise_and(v16, mhalf)

    def fire(j, buf, sem):
        pltpu.async_copy(tab.at[pidx_v.at[j]], buf, sem)

    def drain(j, buf, sem):
        pltpu.make_async_copy(tab.at[pidx_v.at[j]], buf, sem).wait()

    def extract(j, buf):
        for k in range(_CH // 16):
            i16 = lane + j * _CH + k * 16
            ids16 = ids_v[pl.ds(j * _CH + k * 16, 16)]
            off16 = lax.shift_left(
                lax.bitwise_and(
                    lax.shift_right_logical(
                        ids16, _HALF.bit_length() - 1), one16),
                six16)
            iloc16 = lane + k * 16
            for c in range(EMBED_DIM):
                c16 = jnp.full((16,), c, jnp.int32)
                vals = plsc.load_gather(buf, [iloc16, off16 + c16])
                plsc.store_scatter(rows_v, [c16, i16], vals)

    fire(0, pairs0_v, sem0)

    @pl.loop(0, _NCH, step=2)
    def _chunk(g):
        @pl.when(g + 1 < _NCH)
        def _():
            fire(g + 1, pairs1_v, sem1)

        drain(g, pairs0_v, sem0)
        extract(g, pairs0_v)

        @pl.when(g + 2 < _NCH)
        def _():
            fire(g + 2, pairs0_v, sem0)

        @pl.when(g + 1 < _NCH)
        def _():
            drain(g + 1, pairs1_v, sem1)
            extract(g + 1, pairs1_v)

    pltpu.sync_copy(rows_v, out_hbm.at[:, pl.ds(base, _W)])


def kernel(user_ids, pos_item_ids, user_table, item_table):
    u2 = user_ids.astype(jnp.int32).reshape(_NW, _W)
    i2 = pos_item_ids.astype(jnp.int32).reshape(_NW, _W)
    ut2 = _relayout(user_table.T)
    user_embT = _tower_gather(u2, ut2)
    it2 = _relayout(item_table.T)
    item_embT = _tower_gather(i2, it2)
    return (user_embT.T, item_embT.T)
